# baseline (speedup 1.0000x reference)
import jax
import jax.numpy as jnp
from jax import lax
from jax.experimental import pallas as pl
from jax.experimental.pallas import tpu as pltpu
from jax.experimental.pallas import tpu_sc as plsc

NUM_CLUSTERS = 1000000
Z_N = 16
B, L = 4096, 200

NC, NS = 2, 16
NW = NC * NS
BW = B // NW
RG = 4
NG = BW // RG
GSZ = RG * L
SPANS = ((0, 128), (128, 72))


def _gather_body(x_hbm, tab_hbm, out, idx_v, buf, sem):
    wid = lax.axis_index("s") * NC + lax.axis_index("c")
    row0 = wid * BW
    pltpu.sync_copy(x_hbm.at[pl.ds(row0 * L, BW * L)], idx_v)

    def fire(g, b):
        descs = []
        for jr in range(RG):
            r = g * RG + jr
            for c, w in SPANS:
                src_idx = idx_v.at[pl.ds(r * L + c, w)]
                dst = pl.ds(b * GSZ + jr * L + c, w)
                descs.append(
                    pltpu.async_copy(tab_hbm.at[src_idx], buf.at[dst], sem))
        return descs

    def drain(g, b, descs):
        for d in descs:
            d.wait()
        out_sl = pl.ds((row0 + g * RG) * L, GSZ)
        pltpu.sync_copy(buf.at[pl.ds(b * GSZ, GSZ)],
                        out.at[out_sl, pl.ds(0, Z_N)])

    def pair(gg, carry):
        g0 = gg * 2
        d0 = fire(g0, 0)
        d1 = fire(g0 + 1, 1)
        drain(g0, 0, d0)
        drain(g0 + 1, 1, d1)
        return carry

    lax.fori_loop(0, NG // 2, pair, 0)


def _make_gather():
    mesh = plsc.VectorSubcoreMesh(core_axis_name="c", subcore_axis_name="s")
    return pl.kernel(
        _gather_body,
        out_type=jax.ShapeDtypeStruct((B * L, 128), jnp.float32),
        mesh=mesh,
        compiler_params=pltpu.CompilerParams(use_tc_tiling_on_sc=False),
        scratch_types=[
            pltpu.VMEM((BW * L,), jnp.int32),
            pltpu.VMEM((2 * GSZ, Z_N), jnp.float32),
            pltpu.SemaphoreType.DMA,
        ],
    )


def kernel(x, embed_mu, embed_logvar):
    x1 = x.reshape(B * L)
    mu_p = _make_gather()(x1, embed_mu)
    mu = mu_p.reshape(B, L, 128)[:, :, :Z_N]
    pin = (mu[0, 0, 0] * 0.0).astype(jnp.int32)
    lv_p = _make_gather()(x1 + pin, embed_logvar)
    logvar = lv_p.reshape(B, L, 128)[:, :, :Z_N]
    return (mu, mu, logvar)


# speedup vs baseline: 1.0031x; 1.0031x over previous
import jax
import jax.numpy as jnp
from jax import lax
from jax.experimental import pallas as pl
from jax.experimental.pallas import tpu as pltpu
from jax.experimental.pallas import tpu_sc as plsc

NUM_CLUSTERS = 1000000
Z_N = 16
B, L = 4096, 200

NC, NS = 2, 16
NW = NC * NS
BW = B // NW
RG = 4
NG = BW // RG
GSZ = RG * L
SPANS = ((0, 128), (128, 72))


def _gather_body(x_hbm, tab_hbm, out, idx_v, buf, sem):
    wid = lax.axis_index("s") * NC + lax.axis_index("c")
    row0 = wid * BW
    pltpu.sync_copy(x_hbm.at[pl.ds(row0 * L, BW * L)], idx_v)

    def fire(g, b):
        descs = []
        for jr in range(RG):
            r = g * RG + jr
            for c, w in SPANS:
                src_idx = idx_v.at[pl.ds(r * L + c, w)]
                dst = pl.ds(b * GSZ + jr * L + c, w)
                descs.append(
                    pltpu.async_copy(tab_hbm.at[src_idx], buf.at[dst], sem))
        return descs

    def drain(g, b, descs):
        for d in descs:
            d.wait()
        out_sl = pl.ds((row0 + g * RG) * L, GSZ)
        pltpu.sync_copy(buf.at[pl.ds(b * GSZ, GSZ)],
                        out.at[out_sl, pl.ds(0, Z_N)])

    def pair(gg, carry):
        g0 = gg * 2
        d0 = fire(g0, 0)
        d1 = fire(g0 + 1, 1)
        drain(g0, 0, d0)
        drain(g0 + 1, 1, d1)
        return carry

    lax.fori_loop(0, NG // 2, pair, 0)


def _make_gather():
    mesh = plsc.VectorSubcoreMesh(core_axis_name="c", subcore_axis_name="s")
    return pl.kernel(
        _gather_body,
        out_type=jax.ShapeDtypeStruct((B * L, 128), jnp.float32),
        mesh=mesh,
        compiler_params=pltpu.CompilerParams(use_tc_tiling_on_sc=False),
        scratch_types=[
            pltpu.VMEM((BW * L,), jnp.int32),
            pltpu.VMEM((2 * GSZ, Z_N), jnp.float32),
            pltpu.SemaphoreType.DMA,
        ],
    )


def kernel(x, embed_mu, embed_logvar):
    x1 = x.reshape(B * L)
    mu_p = _make_gather()(x1, embed_mu)
    lv_p = _make_gather()(x1, embed_logvar)
    mu = mu_p.reshape(B, L, 128)[:, :, :Z_N]
    logvar = lv_p.reshape(B, L, 128)[:, :, :Z_N]
    return (mu, mu, logvar)
